# baseline (speedup 1.0000x reference)
import jax
import jax.numpy as jnp
from jax import lax
from jax.experimental import pallas as pl
from jax.experimental.pallas import tpu as pltpu

F_DIM = 128
THRESH = 0.1
N = 1024


def _norm_cols(x):
    ss = jnp.sum(x * x, axis=0, keepdims=True)
    return x * (1.0 / jnp.maximum(jnp.sqrt(ss), 1e-8))


def _dg(a, b, ca, cb):
    return lax.dot_general(a, b, (((ca,), (cb,)), ((), ())),
                           preferred_element_type=jnp.float32)


def _hyper_kernel(t_ref, c1_ref, c2_ref, top_ref, botA_ref, botB_ref,
                  w1_ref, b1_ref, w2_ref, b2_ref,
                  ot_ref, oa_ref, ob_ref):
    w1 = w1_ref[:]
    w2 = w2_ref[:]
    b1 = b1_ref[:]
    b2 = b2_ref[:]

    that = _norm_cols(t_ref[0])
    masks = []
    for cref in (c1_ref, c2_ref):
        chat = _norm_cols(cref[0])
        sim = _dg(that, chat, 0, 0)
        masks.append((sim > THRESH).astype(jnp.bfloat16))
    m1, m2 = masks

    xn_top = _dg(w1, top_ref[0], 0, 0) + b1
    xn_botA = (_dg(w1, botA_ref[0], 0, 0) + b1).astype(jnp.bfloat16)
    xn_botB = (_dg(w1, botB_ref[0], 0, 0) + b1).astype(jnp.bfloat16)

    ones_row = jnp.ones((1, N), dtype=jnp.bfloat16)
    deg_e = 1.0 + _dg(ones_row, m1, 1, 1) + _dg(ones_row, m2, 1, 1)
    xe = (xn_top + _dg(xn_botA, m1, 1, 1)
          + _dg(xn_botB, m2, 1, 1)) * (1.0 / deg_e)
    xet = _dg(w2, xe, 0, 0) + b2
    xet_bf = xet.astype(jnp.bfloat16)

    ot_ref[0] = xet
    for m, d in ((m1, oa_ref), (m2, ob_ref)):
        deg_v = jnp.sum(m, axis=0, keepdims=True, dtype=jnp.float32)
        inv_v = 1.0 / jnp.maximum(deg_v, 1.0)
        d[0] = _dg(xet_bf, m, 1, 0) * inv_v


def kernel(X_target, X_context1, X_context2, W1, b1, W2, b2):
    B, C, Hh, Ww = X_target.shape
    n = Hh * Ww
    xt = X_target.reshape(B, C, n)
    xc1 = X_context1.reshape(B, C, n)
    xc2 = X_context2.reshape(B, C, n)

    top = jnp.stack([xt[0], xc2[0]])
    botA = jnp.stack([xt[1], xc1[1]])
    botB = jnp.stack([xc1[0], xc2[1]])

    blk = pl.BlockSpec((1, C, n), lambda i: (i, 0, 0))
    wspec = pl.BlockSpec((C, C), lambda i: (0, 0))
    bspec = pl.BlockSpec((C, 1), lambda i: (0, 0))
    out_sd = jax.ShapeDtypeStruct((B, C, n), jnp.float32)
    ot, oa, ob = pl.pallas_call(
        _hyper_kernel,
        grid=(B,),
        in_specs=[blk, blk, blk, blk, blk, blk, wspec, bspec, wspec, bspec],
        out_specs=(blk, blk, blk),
        out_shape=(out_sd, out_sd, out_sd),
        compiler_params=pltpu.CompilerParams(
            dimension_semantics=("parallel",)),
    )(xt, xc1, xc2, top, botA, botB,
      W1, b1.reshape(C, 1), W2, b2.reshape(C, 1))

    shp = (B, C, Hh, Ww)
    t_out = jnp.stack([ot[0], oa[0]]).reshape(shp)
    c1_out = jnp.stack([ob[0], oa[1]]).reshape(shp)
    c2_out = jnp.stack([ot[1], ob[1]]).reshape(shp)
    return (t_out, c1_out, c2_out)


# speedup vs baseline: 1.2754x; 1.2754x over previous
import jax
import jax.numpy as jnp
from jax import lax
from jax.experimental import pallas as pl

F_DIM = 128
THRESH = 0.1
B = 2
N = 1024
N_CTX = 2048
S = N + N_CTX
V = B * S


def _hyper_kernel(x_ref, w1_ref, b1_ref, w2_ref, b2_ref, out_ref):
    x = x_ref[:]
    w1 = w1_ref[:]
    w2 = w2_ref[:]
    b1 = b1_ref[:]
    b2 = b2_ref[:]

    nrm = jnp.sqrt(jnp.sum(x * x, axis=1, keepdims=True))
    xhat = x / jnp.maximum(nrm, 1e-8)

    xn = jnp.dot(x, w1, preferred_element_type=jnp.float32) + b1

    for i in range(B):
        xt_n = xhat[i * N:(i + 1) * N]
        xc_n = xhat[B * N + i * N_CTX:B * N + (i + 1) * N_CTX]
        sim = lax.dot_general(
            xt_n, xc_n, (((1,), (1,)), ((), ())),
            preferred_element_type=jnp.float32)
        m = (sim > THRESH).astype(jnp.float32)

        y = xn[i * S:(i + 1) * S]
        deg_e = 1.0 + jnp.sum(m, axis=1, keepdims=True)
        xe = (y[:N] + jnp.dot(m, y[N:], preferred_element_type=jnp.float32)) / deg_e
        xet = jnp.dot(xe, w2, preferred_element_type=jnp.float32) + b2

        out_ref[i * S:i * S + N, :] = xet
        deg_v = jnp.maximum(jnp.sum(m, axis=0, keepdims=True), 1.0)
        bot = lax.dot_general(
            m, xet, (((0,), (0,)), ((), ())),
            preferred_element_type=jnp.float32)
        out_ref[i * S + N:(i + 1) * S, :] = bot / deg_v.T


def kernel(X_target, X_context1, X_context2, W1, b1, W2, b2):
    Bb, C, Hh, Ww = X_target.shape
    n = Hh * Ww
    to_rows = lambda a: jnp.transpose(a, (0, 2, 3, 1)).reshape(Bb * n, C)
    Xt = to_rows(X_target)
    Xc1 = jnp.transpose(X_context1, (0, 2, 3, 1)).reshape(Bb, n, C)
    Xc2 = jnp.transpose(X_context2, (0, 2, 3, 1)).reshape(Bb, n, C)
    Xc = jnp.concatenate([Xc1, Xc2], axis=1).reshape(Bb * 2 * n, C)
    x_all = jnp.concatenate([Xt, Xc], axis=0)

    x_new = pl.pallas_call(
        _hyper_kernel,
        out_shape=jax.ShapeDtypeStruct((V, F_DIM), jnp.float32),
    )(x_all, W1, b1.reshape(1, F_DIM), W2, b2.reshape(1, F_DIM))

    to_nchw = lambda a: jnp.transpose(a, (0, 3, 1, 2))
    xt_out = to_nchw(x_new[:Bb * n].reshape(Bb, Hh, Ww, C))
    xc_out = x_new[Bb * n:].reshape(Bb, 2 * n, C)
    xc1_out = to_nchw(xc_out[:, :n, :].reshape(Bb, Hh, Ww, C))
    xc2_out = to_nchw(xc_out[:, n:, :].reshape(Bb, Hh, Ww, C))
    return (xt_out, xc1_out, xc2_out)
